# Initial kernel scaffold; baseline (speedup 1.0000x reference)
import jax
import jax.numpy as jnp
from jax.experimental import pallas as pl


def kernel(inputs, token_table, pos_table):
    raise NotImplementedError("write your pallas kernel here")


# speedup vs baseline: 2.6819x; 2.6819x over previous
import jax
import jax.numpy as jnp
from jax import lax
from jax.experimental import pallas as pl
from jax.experimental.pallas import tpu as pltpu
from jax.experimental.pallas import tpu_sc as plsc

SEQ_LEN = 200
DIM = 128
BATCH = 4096
LANES = 16

NUM_CORES = 2
NUM_SUBCORES = 16
NUM_WORKERS = NUM_CORES * NUM_SUBCORES

CHUNK = 128
TOKENS = BATCH * SEQ_LEN
ROWS_PER_W = TOKENS // NUM_WORKERS
CHUNKS_PER_W = ROWS_PER_W // CHUNK
VECS_PER_ROW = DIM // LANES


def _body(idx_hbm, table_hbm, pos_hbm, out_hbm, idx_v, pos_v, buf0, buf1,
          sem0, sem1):
  c = lax.axis_index("c")
  s = lax.axis_index("s")
  wid = s * NUM_CORES + c

  pltpu.sync_copy(pos_hbm, pos_v)
  pltpu.sync_copy(idx_hbm.at[pl.ds(wid * CHUNKS_PER_W, CHUNKS_PER_W)], idx_v)

  row0 = wid * ROWS_PER_W
  bufs = (buf0, buf1)
  sems = (sem0, sem1)

  pltpu.async_copy(table_hbm.at[idx_v.at[0]], buf0, sem0)
  pltpu.async_copy(table_hbm.at[idx_v.at[1]], buf1, sem1)

  def outer(g, _):
    for p in range(2):
      gg = g * 2 + p
      buf = bufs[p]
      pltpu.make_async_copy(table_hbm.at[idx_v.at[gg]], buf, sems[p]).wait()

      pbase = lax.rem(gg * CHUNK, SEQ_LEN)

      def add_row(r, _):
        pr = pbase + r
        pr = jnp.where(pr >= SEQ_LEN, pr - SEQ_LEN, pr)
        for v in range(VECS_PER_ROW):
          sl = pl.ds(v * LANES, LANES)
          buf[r, sl] = buf[r, sl] + pos_v[pr, sl]
        return _

      lax.fori_loop(0, CHUNK, add_row, None, unroll=2)

      pltpu.sync_copy(buf, out_hbm.at[pl.ds(row0 + gg * CHUNK, CHUNK)])

      @pl.when(gg + 2 < CHUNKS_PER_W)
      def _start():
        pltpu.async_copy(table_hbm.at[idx_v.at[gg + 2]], buf, sems[p])

    return _

  lax.fori_loop(0, CHUNKS_PER_W // 2, outer, None)


@jax.jit
def _run(idx2d, token_table, pos_table):
  mesh = plsc.VectorSubcoreMesh(core_axis_name="c", subcore_axis_name="s")
  f = pl.kernel(
      _body,
      out_type=jax.ShapeDtypeStruct((TOKENS, DIM), jnp.float32),
      mesh=mesh,
      scratch_types=[
          pltpu.VMEM((CHUNKS_PER_W, CHUNK), jnp.int32),
          pltpu.VMEM((SEQ_LEN, DIM), jnp.float32),
          pltpu.VMEM((CHUNK, DIM), jnp.float32),
          pltpu.VMEM((CHUNK, DIM), jnp.float32),
          pltpu.SemaphoreType.DMA,
          pltpu.SemaphoreType.DMA,
      ],
  )
  return f(idx2d, token_table, pos_table)


def kernel(inputs, token_table, pos_table):
  idx2d = inputs.astype(jnp.int32).reshape(TOKENS // CHUNK, CHUNK)
  out = _run(idx2d, token_table, pos_table)
  return out.reshape(BATCH, SEQ_LEN, DIM)


# speedup vs baseline: 7.2835x; 2.7158x over previous
import jax
import jax.numpy as jnp
from jax import lax
from jax.experimental import pallas as pl
from jax.experimental.pallas import tpu as pltpu
from jax.experimental.pallas import tpu_sc as plsc

SEQ_LEN = 200
DIM = 128
BATCH = 4096
LANES = 16

NUM_CORES = 2
NUM_SUBCORES = 16
NUM_WORKERS = NUM_CORES * NUM_SUBCORES

CHUNK = 128
TOKENS = BATCH * SEQ_LEN
ROWS_PER_W = TOKENS // NUM_WORKERS
CHUNKS_PER_W = ROWS_PER_W // CHUNK
VECS_PER_ROW = DIM // LANES


def _body(idx_hbm, table_hbm, pos_hbm, out_hbm, idx_v, pos_v, buf0, buf1,
          sem0, sem1):
  c = lax.axis_index("c")
  s = lax.axis_index("s")
  wid = s * NUM_CORES + c

  pltpu.sync_copy(pos_hbm, pos_v.at[pl.ds(0, SEQ_LEN)])
  pltpu.sync_copy(pos_hbm, pos_v.at[pl.ds(SEQ_LEN, SEQ_LEN)])
  pltpu.sync_copy(idx_hbm.at[pl.ds(wid * CHUNKS_PER_W, CHUNKS_PER_W)], idx_v)

  row0 = wid * ROWS_PER_W
  bufs = (buf0, buf1)
  sems = (sem0, sem1)

  pltpu.async_copy(table_hbm.at[idx_v.at[0]], buf0, sem0)
  pltpu.async_copy(table_hbm.at[idx_v.at[1]], buf1, sem1)

  def outer(g, _):
    for p in range(2):
      gg = g * 2 + p
      buf = bufs[p]
      pltpu.make_async_copy(table_hbm.at[idx_v.at[gg]], buf, sems[p]).wait()

      pbase = lax.rem(gg * CHUNK, SEQ_LEN)

      @plsc.parallel_loop(0, CHUNK, 1, unroll=4)
      def _add_row(r):
        pr = pbase + r
        for v in range(VECS_PER_ROW):
          sl = pl.ds(v * LANES, LANES)
          buf[r, sl] = buf[r, sl] + pos_v[pr, sl]

      pltpu.sync_copy(buf, out_hbm.at[pl.ds(row0 + gg * CHUNK, CHUNK)])

      @pl.when(gg + 2 < CHUNKS_PER_W)
      def _start():
        pltpu.async_copy(table_hbm.at[idx_v.at[gg + 2]], buf, sems[p])

    return _

  lax.fori_loop(0, CHUNKS_PER_W // 2, outer, None)


@jax.jit
def _run(idx2d, token_table, pos_table):
  mesh = plsc.VectorSubcoreMesh(core_axis_name="c", subcore_axis_name="s")
  f = pl.kernel(
      _body,
      out_type=jax.ShapeDtypeStruct((TOKENS, DIM), jnp.float32),
      mesh=mesh,
      scratch_types=[
          pltpu.VMEM((CHUNKS_PER_W, CHUNK), jnp.int32),
          pltpu.VMEM((2 * SEQ_LEN, DIM), jnp.float32),
          pltpu.VMEM((CHUNK, DIM), jnp.float32),
          pltpu.VMEM((CHUNK, DIM), jnp.float32),
          pltpu.SemaphoreType.DMA,
          pltpu.SemaphoreType.DMA,
      ],
  )
  return f(idx2d, token_table, pos_table)


def kernel(inputs, token_table, pos_table):
  idx2d = inputs.astype(jnp.int32).reshape(TOKENS // CHUNK, CHUNK)
  out = _run(idx2d, token_table, pos_table)
  return out.reshape(BATCH, SEQ_LEN, DIM)


# speedup vs baseline: 8.3800x; 1.1505x over previous
import jax
import jax.numpy as jnp
from jax import lax
from jax.experimental import pallas as pl
from jax.experimental.pallas import tpu as pltpu
from jax.experimental.pallas import tpu_sc as plsc

SEQ_LEN = 200
DIM = 128
BATCH = 4096
LANES = 16

NUM_CORES = 2
NUM_SUBCORES = 16
NUM_WORKERS = NUM_CORES * NUM_SUBCORES

CHUNK = 128
TOKENS = BATCH * SEQ_LEN
ROWS_PER_W = TOKENS // NUM_WORKERS
CHUNKS_PER_W = ROWS_PER_W // CHUNK
VECS_PER_ROW = DIM // LANES


def _body(idx_hbm, table_hbm, pos_hbm, out_hbm, idx_v, pos_v, buf0, buf1,
          buf2, gsem0, gsem1, gsem2, ssem0, ssem1, ssem2):
  c = lax.axis_index("c")
  s = lax.axis_index("s")
  wid = s * NUM_CORES + c

  pltpu.sync_copy(pos_hbm, pos_v.at[pl.ds(0, SEQ_LEN)])
  pltpu.sync_copy(pos_hbm, pos_v.at[pl.ds(SEQ_LEN, SEQ_LEN)])
  pltpu.sync_copy(idx_hbm.at[pl.ds(wid * CHUNKS_PER_W, CHUNKS_PER_W)], idx_v)

  row0 = wid * ROWS_PER_W
  bufs = (buf0, buf1, buf2)
  gsems = (gsem0, gsem1, gsem2)
  ssems = (ssem0, ssem1, ssem2)
  NBUF = 3

  def out_slice(gg):
    return out_hbm.at[pl.ds(row0 + gg * CHUNK, CHUNK)]

  pltpu.async_copy(table_hbm.at[idx_v.at[0]], buf0, gsem0)
  pltpu.async_copy(table_hbm.at[idx_v.at[1]], buf1, gsem1)

  def process(gg, p):
    buf = bufs[p]
    q = (p + 2) % NBUF
    pltpu.make_async_copy(table_hbm.at[idx_v.at[gg]], buf, gsems[p]).wait()

    pbase = lax.rem(gg * CHUNK, SEQ_LEN)

    @plsc.parallel_loop(0, CHUNK, 1, unroll=4)
    def _add_row(r):
      pr = pbase + r
      for v in range(VECS_PER_ROW):
        sl = pl.ds(v * LANES, LANES)
        buf[r, sl] = buf[r, sl] + pos_v[pr, sl]

    @pl.when(gg >= 1)
    def _drain():
      pltpu.make_async_copy(bufs[q], out_slice(gg - 1), ssems[q]).wait()

    @pl.when(gg + 2 < CHUNKS_PER_W)
    def _start():
      pltpu.async_copy(table_hbm.at[idx_v.at[gg + 2]], bufs[q], gsems[q])

    pltpu.async_copy(buf, out_slice(gg), ssems[p])

  def outer(g, _):
    for p in range(NBUF):
      process(g * NBUF + p, p)
    return _

  n_main = CHUNKS_PER_W // NBUF * NBUF
  lax.fori_loop(0, CHUNKS_PER_W // NBUF, outer, None)

  for gg in range(n_main, CHUNKS_PER_W):
    process(gg, gg % NBUF)
  last = CHUNKS_PER_W - 1
  pltpu.make_async_copy(bufs[last % NBUF], out_slice(last),
                        ssems[last % NBUF]).wait()


@jax.jit
def _run(idx2d, token_table, pos_table):
  mesh = plsc.VectorSubcoreMesh(core_axis_name="c", subcore_axis_name="s")
  f = pl.kernel(
      _body,
      out_type=jax.ShapeDtypeStruct((TOKENS, DIM), jnp.float32),
      mesh=mesh,
      scratch_types=[
          pltpu.VMEM((CHUNKS_PER_W, CHUNK), jnp.int32),
          pltpu.VMEM((2 * SEQ_LEN, DIM), jnp.float32),
          pltpu.VMEM((CHUNK, DIM), jnp.float32),
          pltpu.VMEM((CHUNK, DIM), jnp.float32),
          pltpu.VMEM((CHUNK, DIM), jnp.float32),
          pltpu.SemaphoreType.DMA,
          pltpu.SemaphoreType.DMA,
          pltpu.SemaphoreType.DMA,
          pltpu.SemaphoreType.DMA,
          pltpu.SemaphoreType.DMA,
          pltpu.SemaphoreType.DMA,
      ],
  )
  return f(idx2d, token_table, pos_table)


def kernel(inputs, token_table, pos_table):
  idx2d = inputs.astype(jnp.int32).reshape(TOKENS // CHUNK, CHUNK)
  out = _run(idx2d, token_table, pos_table)
  return out.reshape(BATCH, SEQ_LEN, DIM)


# speedup vs baseline: 8.9518x; 1.0682x over previous
import jax
import jax.numpy as jnp
from jax import lax
from jax.experimental import pallas as pl
from jax.experimental.pallas import tpu as pltpu
from jax.experimental.pallas import tpu_sc as plsc

SEQ_LEN = 200
DIM = 128
BATCH = 4096
LANES = 16

NUM_CORES = 2
NUM_SUBCORES = 16
NUM_WORKERS = NUM_CORES * NUM_SUBCORES

CHUNK = 128
TOKENS = BATCH * SEQ_LEN
ROWS_PER_W = TOKENS // NUM_WORKERS
CHUNKS_PER_W = ROWS_PER_W // CHUNK
VECS_PER_ROW = DIM // LANES


def _body(idx_hbm, table_hbm, pos_hbm, out_hbm, idx_v, pos_v, buf0, buf1,
          buf2, gsem0, gsem1, gsem2, ssem0, ssem1, ssem2):
  c = lax.axis_index("c")
  s = lax.axis_index("s")
  wid = s * NUM_CORES + c

  pltpu.sync_copy(pos_hbm, pos_v.at[pl.ds(0, SEQ_LEN)])
  pltpu.sync_copy(pos_hbm, pos_v.at[pl.ds(SEQ_LEN, SEQ_LEN)])
  pltpu.sync_copy(idx_hbm.at[pl.ds(wid * CHUNKS_PER_W, CHUNKS_PER_W)], idx_v)

  row0 = wid * ROWS_PER_W
  bufs = (buf0, buf1, buf2)
  gsems = (gsem0, gsem1, gsem2)
  ssems = (ssem0, ssem1, ssem2)
  NBUF = 3

  def out_slice(gg):
    return out_hbm.at[pl.ds(row0 + gg * CHUNK, CHUNK)]

  def prefill(gg, buf):
    pbase = lax.rem(gg * CHUNK, SEQ_LEN)

    @plsc.parallel_loop(0, CHUNK, 1, unroll=4)
    def _copy_row(r):
      pr = pbase + r
      for v in range(VECS_PER_ROW):
        sl = pl.ds(v * LANES, LANES)
        buf[r, sl] = pos_v[pr, sl]

  prefill(0, buf0)
  pltpu.async_copy(table_hbm.at[idx_v.at[0]], buf0, gsem0, add=True)
  prefill(1, buf1)
  pltpu.async_copy(table_hbm.at[idx_v.at[1]], buf1, gsem1, add=True)

  def process(gg, p):
    buf = bufs[p]
    q = (p + 2) % NBUF
    pltpu.make_async_copy(table_hbm.at[idx_v.at[gg]], buf, gsems[p]).wait()

    @pl.when(gg >= 1)
    def _drain():
      pltpu.make_async_copy(bufs[q], out_slice(gg - 1), ssems[q]).wait()

    @pl.when(gg + 2 < CHUNKS_PER_W)
    def _start():
      prefill(gg + 2, bufs[q])
      pltpu.async_copy(table_hbm.at[idx_v.at[gg + 2]], bufs[q], gsems[q],
                       add=True)

    pltpu.async_copy(buf, out_slice(gg), ssems[p])

  def outer(g, _):
    for p in range(NBUF):
      process(g * NBUF + p, p)
    return _

  n_main = CHUNKS_PER_W // NBUF * NBUF
  lax.fori_loop(0, CHUNKS_PER_W // NBUF, outer, None)

  for gg in range(n_main, CHUNKS_PER_W):
    process(gg, gg % NBUF)
  last = CHUNKS_PER_W - 1
  pltpu.make_async_copy(bufs[last % NBUF], out_slice(last),
                        ssems[last % NBUF]).wait()


@jax.jit
def _run(idx2d, token_table, pos_table):
  mesh = plsc.VectorSubcoreMesh(core_axis_name="c", subcore_axis_name="s")
  f = pl.kernel(
      _body,
      out_type=jax.ShapeDtypeStruct((TOKENS, DIM), jnp.float32),
      mesh=mesh,
      scratch_types=[
          pltpu.VMEM((CHUNKS_PER_W, CHUNK), jnp.int32),
          pltpu.VMEM((2 * SEQ_LEN, DIM), jnp.float32),
          pltpu.VMEM((CHUNK, DIM), jnp.float32),
          pltpu.VMEM((CHUNK, DIM), jnp.float32),
          pltpu.VMEM((CHUNK, DIM), jnp.float32),
          pltpu.SemaphoreType.DMA,
          pltpu.SemaphoreType.DMA,
          pltpu.SemaphoreType.DMA,
          pltpu.SemaphoreType.DMA,
          pltpu.SemaphoreType.DMA,
          pltpu.SemaphoreType.DMA,
      ],
  )
  return f(idx2d, token_table, pos_table)


def kernel(inputs, token_table, pos_table):
  idx2d = inputs.astype(jnp.int32).reshape(TOKENS // CHUNK, CHUNK)
  out = _run(idx2d, token_table, pos_table)
  return out.reshape(BATCH, SEQ_LEN, DIM)
